# baseline (speedup 1.0000x reference)
import functools

import jax
import jax.numpy as jnp
from jax import lax
from jax.experimental import pallas as pl
from jax.experimental.pallas import tpu as pltpu
from jax.experimental.pallas import tpu_sc as plsc

_NC = 2
_NS = 16
_NW = _NC * _NS
_L = 16
_PAD = 16


def _combine_body(bias_ref, fc_ref, emb_ref, out_ref):
    e = emb_ref[...]
    c = (fc_ref[...] - 0.5 * jnp.sum(e * e, axis=1, keepdims=True)
         + bias_ref[0, 0])
    out_ref[...] = jnp.broadcast_to(c, (c.shape[0], _PAD))


def _make_combined_table(fc_weight, embedding_weight, bias, n_fields):
    v, d = embedding_weight.shape
    rb = 1000
    assert v % rb == 0
    bias_over_f = (bias.astype(jnp.float32) / n_fields).reshape(1, 1)
    return pl.pallas_call(
        _combine_body,
        grid=(v // rb,),
        in_specs=[
            pl.BlockSpec(memory_space=pltpu.SMEM),
            pl.BlockSpec((rb, 1), lambda i: (i, 0)),
            pl.BlockSpec((rb, d), lambda i: (i, 0)),
        ],
        out_specs=pl.BlockSpec((rb, _PAD), lambda i: (i, 0)),
        out_shape=jax.ShapeDtypeStruct((v, _PAD), jnp.float32),
    )(bias_over_f, fc_weight, embedding_weight)


def _gather_add_kernel(x3, tab, *, batch, n_fields, width):
    bpw = batch // _NW
    nch = bpw // 128
    mesh = plsc.VectorSubcoreMesh(
        core_axis_name="c", subcore_axis_name="s",
        num_cores=_NC, num_subcores=_NS,
    )

    @functools.partial(
        pl.kernel,
        out_type=jax.ShapeDtypeStruct((batch, width), jnp.float32),
        mesh=mesh,
        compiler_params=pltpu.CompilerParams(use_tc_tiling_on_sc=False),
        scratch_types=[
            pltpu.VMEM((n_fields, nch, 128), jnp.int32),
            pltpu.VMEM((bpw, width), jnp.float32),
            pltpu.SemaphoreType.DMA,
        ],
    )
    def body(x3_hbm, tab_hbm, out_hbm, idx_v, acc_v, sem):
        cid = lax.axis_index("c")
        sid = lax.axis_index("s")
        wid = sid * _NC + cid
        base = wid * bpw
        cb = wid * nch

        idx_cps = [
            pltpu.async_copy(x3_hbm.at[f, pl.ds(cb, nch), :], idx_v.at[f], sem)
            for f in range(n_fields)
        ]

        zeros = jnp.zeros((_L,), jnp.float32)

        def zero_row(i, carry):
            for j in range(width // _L):
                acc_v[i, pl.ds(j * _L, _L)] = zeros
            return carry

        lax.fori_loop(0, bpw, zero_row, 0)
        for cp in idx_cps:
            cp.wait()

        def gather_field(f, carry):
            cps = [
                pltpu.async_copy(
                    tab_hbm.at[idx_v.at[f, ch]],
                    acc_v.at[pl.ds(ch * 128, 128)],
                    sem, add=True)
                for ch in range(nch)
            ]
            for cp in cps:
                cp.wait()
            return carry

        lax.fori_loop(0, n_fields, gather_field, 0)

        pltpu.sync_copy(acc_v, out_hbm.at[pl.ds(base, bpw), :])

    return body(x3, tab)


def _epilogue_body(s_ref, zacc_ref, out_ref):
    s = s_ref[...]
    z = zacc_ref[:, :1] + 0.5 * jnp.sum(s * s, axis=1, keepdims=True)
    out_ref[...] = 1.0 / (1.0 + jnp.exp(-z))


def _epilogue(s, zacc, *, batch, d):
    rb = 2048
    assert batch % rb == 0
    out = pl.pallas_call(
        _epilogue_body,
        grid=(batch // rb,),
        in_specs=[
            pl.BlockSpec((rb, d), lambda i: (i, 0)),
            pl.BlockSpec((rb, _PAD), lambda i: (i, 0)),
        ],
        out_specs=pl.BlockSpec((rb, 1), lambda i: (i, 0)),
        out_shape=jax.ShapeDtypeStruct((batch, 1), jnp.float32),
    )(s, zacc)
    return out.reshape(batch)


def kernel(x, fc_weight, bias, embedding_weight):
    batch, n_fields = x.shape
    v, d = embedding_weight.shape
    assert batch % (_NW * 128) == 0 and d % _L == 0

    x3 = x.astype(jnp.int32).T.reshape(n_fields, batch // 128, 128)
    s = _gather_add_kernel(x3, embedding_weight,
                           batch=batch, n_fields=n_fields, width=d)
    c_tab = _make_combined_table(fc_weight, embedding_weight, bias, n_fields)
    zacc = _gather_add_kernel(x3, c_tab,
                              batch=batch, n_fields=n_fields, width=_PAD)
    return _epilogue(s, zacc, batch=batch, d=d)


# speedup vs baseline: 1.3254x; 1.3254x over previous
import functools

import jax
import jax.numpy as jnp
from jax import lax
from jax.experimental import pallas as pl
from jax.experimental.pallas import tpu as pltpu
from jax.experimental.pallas import tpu_sc as plsc

_NC = 2
_NS = 16
_NW = _NC * _NS
_L = 16
_PAD = 16


def _combine_body(bias_ref, fc_ref, emb_ref, out_ref):
    e = emb_ref[...]
    out_ref[...] = (fc_ref[...] - 0.5 * jnp.sum(e * e, axis=1)
                    + bias_ref[0, 0])


def _make_combined_table(fc_weight, embedding_weight, bias, n_fields):
    v, d = embedding_weight.shape
    rb = 1024
    n_blocks = -(-v // rb)
    v_pad = n_blocks * rb
    bias_over_f = (bias.astype(jnp.float32) / n_fields).reshape(1, 1)
    return pl.pallas_call(
        _combine_body,
        grid=(n_blocks,),
        in_specs=[
            pl.BlockSpec(memory_space=pltpu.SMEM),
            pl.BlockSpec((rb,), lambda i: (i,)),
            pl.BlockSpec((rb, d), lambda i: (i, 0)),
        ],
        out_specs=pl.BlockSpec((rb,), lambda i: (i,)),
        out_shape=jax.ShapeDtypeStruct((v_pad,), jnp.float32),
    )(bias_over_f, fc_weight.reshape(v), embedding_weight)


def _expand_c_kernel(c, *, v):
    rows_pw = v // _NW
    assert rows_pw % 8 == 0
    mesh = plsc.VectorSubcoreMesh(
        core_axis_name="c", subcore_axis_name="s",
        num_cores=_NC, num_subcores=_NS,
    )

    @functools.partial(
        pl.kernel,
        out_type=jax.ShapeDtypeStruct((v, _PAD), jnp.float32),
        mesh=mesh,
        compiler_params=pltpu.CompilerParams(use_tc_tiling_on_sc=False),
        scratch_types=[
            pltpu.VMEM((rows_pw,), jnp.float32),
            pltpu.VMEM((rows_pw, _PAD), jnp.float32),
            pltpu.SemaphoreType.DMA,
        ],
    )
    def body(c_hbm, out_hbm, c_v, e_v, sem):
        cid = lax.axis_index("c")
        sid = lax.axis_index("s")
        wid = sid * _NC + cid
        base = wid * rows_pw

        pltpu.async_copy(c_hbm.at[pl.ds(base, rows_pw)], c_v, sem).wait()

        assert rows_pw % _L == 0

        def expand_group(g, carry):
            vals = c_v[pl.ds(g * _L, _L)]
            for k in range(_L):
                e_v[g * _L + k, pl.ds(0, _L)] = jnp.full(
                    (_L,), vals[k], jnp.float32)
            return carry

        lax.fori_loop(0, rows_pw // _L, expand_group, 0)
        pltpu.sync_copy(e_v, out_hbm.at[pl.ds(base, rows_pw), :])

    return body(c)


def _fm_sc_kernel(x3, c_tab, emb, *, batch, n_fields, d):
    bpw = batch // _NW
    nch = bpw // 128
    mesh = plsc.VectorSubcoreMesh(
        core_axis_name="c", subcore_axis_name="s",
        num_cores=_NC, num_subcores=_NS,
    )

    @functools.partial(
        pl.kernel,
        out_type=(
            jax.ShapeDtypeStruct((batch, d), jnp.float32),
            jax.ShapeDtypeStruct((batch, _PAD), jnp.float32),
        ),
        mesh=mesh,
        compiler_params=pltpu.CompilerParams(use_tc_tiling_on_sc=False),
        scratch_types=[
            pltpu.VMEM((n_fields, nch, 128), jnp.int32),
            pltpu.VMEM((bpw, d), jnp.float32),
            pltpu.VMEM((bpw, _PAD), jnp.float32),
            pltpu.SemaphoreType.DMA,
        ],
    )
    def body(x3_hbm, c_hbm, emb_hbm, s_hbm, zacc_hbm, idx_v, s_v, acc_v, sem):
        cid = lax.axis_index("c")
        sid = lax.axis_index("s")
        wid = sid * _NC + cid
        base = wid * bpw
        cb = wid * nch

        idx_cps = [
            pltpu.async_copy(x3_hbm.at[f, pl.ds(cb, nch), :], idx_v.at[f], sem)
            for f in range(n_fields)
        ]

        zeros = jnp.zeros((_L,), jnp.float32)

        def zero_row(i, carry):
            for j in range(d // _L):
                s_v[i, pl.ds(j * _L, _L)] = zeros
            return carry

        lax.fori_loop(0, bpw, zero_row, 0)

        def zero_acc(i, carry):
            acc_v[i, pl.ds(0, _L)] = zeros
            return carry

        lax.fori_loop(0, bpw, zero_acc, 0)
        for cp in idx_cps:
            cp.wait()

        def gather_field(f, carry):
            cps = []
            for ch in range(nch):
                cps.append(pltpu.async_copy(
                    emb_hbm.at[idx_v.at[f, ch]],
                    s_v.at[pl.ds(ch * 128, 128)],
                    sem, add=True))
                cps.append(pltpu.async_copy(
                    c_hbm.at[idx_v.at[f, ch]],
                    acc_v.at[pl.ds(ch * 128, 128)],
                    sem, add=True))
            for cp in cps:
                cp.wait()
            return carry

        lax.fori_loop(0, n_fields, gather_field, 0)

        pltpu.sync_copy(s_v, s_hbm.at[pl.ds(base, bpw), :])
        pltpu.sync_copy(acc_v, zacc_hbm.at[pl.ds(base, bpw), :])

    return body(x3, c_tab, emb)


def _epilogue_body(s_ref, zacc_ref, out_ref):
    s = s_ref[...]
    z = zacc_ref[:, :1] + 0.5 * jnp.sum(s * s, axis=1, keepdims=True)
    out_ref[...] = 1.0 / (1.0 + jnp.exp(-z))


def _epilogue(s, zacc, *, batch, d):
    rb = 2048
    assert batch % rb == 0
    out = pl.pallas_call(
        _epilogue_body,
        grid=(batch // rb,),
        in_specs=[
            pl.BlockSpec((rb, d), lambda i: (i, 0)),
            pl.BlockSpec((rb, _PAD), lambda i: (i, 0)),
        ],
        out_specs=pl.BlockSpec((rb, 1), lambda i: (i, 0)),
        out_shape=jax.ShapeDtypeStruct((batch, 1), jnp.float32),
    )(s, zacc)
    return out.reshape(batch)


def kernel(x, fc_weight, bias, embedding_weight):
    batch, n_fields = x.shape
    v, d = embedding_weight.shape
    assert batch % (_NW * 128) == 0 and d % _L == 0

    c_pad = _make_combined_table(fc_weight, embedding_weight, bias, n_fields)
    v_pad = c_pad.shape[0]
    assert v_pad % (_NW * 8) == 0
    c_tab = _expand_c_kernel(c_pad, v=v_pad)
    x3 = x.astype(jnp.int32).T.reshape(n_fields, batch // 128, 128)
    s, zacc = _fm_sc_kernel(x3, c_tab, embedding_weight,
                            batch=batch, n_fields=n_fields, d=d)
    return _epilogue(s, zacc, batch=batch, d=d)


# speedup vs baseline: 1.5312x; 1.1553x over previous
import functools

import jax
import jax.numpy as jnp
from jax import lax
from jax.experimental import pallas as pl
from jax.experimental.pallas import tpu as pltpu
from jax.experimental.pallas import tpu_sc as plsc

_NC = 2
_NS = 16
_NW = _NC * _NS
_L = 16
_PAD = 16


def _combine_body(bias_ref, fc_ref, emb_ref, out_ref):
    e = emb_ref[...]
    sq = jnp.dot(e * e, jnp.ones((e.shape[1],), jnp.float32),
                 preferred_element_type=jnp.float32)
    out_ref[...] = fc_ref[...] - 0.5 * sq + bias_ref[0, 0]


def _make_combined_table(fc_weight, embedding_weight, bias, n_fields):
    v, d = embedding_weight.shape
    rb = 4096
    n_blocks = -(-v // rb)
    v_pad = n_blocks * rb
    bias_over_f = (bias.astype(jnp.float32) / n_fields).reshape(1, 1)
    return pl.pallas_call(
        _combine_body,
        grid=(n_blocks,),
        in_specs=[
            pl.BlockSpec(memory_space=pltpu.SMEM),
            pl.BlockSpec((rb,), lambda i: (i,)),
            pl.BlockSpec((rb, d), lambda i: (i, 0)),
        ],
        out_specs=pl.BlockSpec((rb,), lambda i: (i,)),
        out_shape=jax.ShapeDtypeStruct((v_pad,), jnp.float32),
    )(bias_over_f, fc_weight.reshape(v), embedding_weight)


def _expand_c_kernel(c, *, v):
    rows_pw = v // _NW
    assert rows_pw % 8 == 0
    mesh = plsc.VectorSubcoreMesh(
        core_axis_name="c", subcore_axis_name="s",
        num_cores=_NC, num_subcores=_NS,
    )

    @functools.partial(
        pl.kernel,
        out_type=jax.ShapeDtypeStruct((v, _PAD), jnp.float32),
        mesh=mesh,
        compiler_params=pltpu.CompilerParams(use_tc_tiling_on_sc=False),
        scratch_types=[
            pltpu.VMEM((rows_pw,), jnp.float32),
            pltpu.VMEM((rows_pw, _PAD), jnp.float32),
            pltpu.SemaphoreType.DMA,
        ],
    )
    def body(c_hbm, out_hbm, c_v, e_v, sem):
        cid = lax.axis_index("c")
        sid = lax.axis_index("s")
        wid = sid * _NC + cid
        base = wid * rows_pw

        pltpu.async_copy(c_hbm.at[pl.ds(base, rows_pw)], c_v, sem).wait()

        assert rows_pw % _L == 0

        def expand_group(g, carry):
            vals = c_v[pl.ds(g * _L, _L)]
            for k in range(_L):
                e_v[g * _L + k, pl.ds(0, _L)] = jnp.full(
                    (_L,), vals[k], jnp.float32)
            return carry

        lax.fori_loop(0, rows_pw // _L, expand_group, 0)
        pltpu.sync_copy(e_v, out_hbm.at[pl.ds(base, rows_pw), :])

    return body(c)


def _fm_sc_kernel(x3, c_tab, emb, *, batch, n_fields, d):
    bpw = batch // _NW
    nch = bpw // 128
    mesh = plsc.VectorSubcoreMesh(
        core_axis_name="c", subcore_axis_name="s",
        num_cores=_NC, num_subcores=_NS,
    )

    @functools.partial(
        pl.kernel,
        out_type=(
            jax.ShapeDtypeStruct((batch, d), jnp.float32),
            jax.ShapeDtypeStruct((batch, _PAD), jnp.float32),
        ),
        mesh=mesh,
        compiler_params=pltpu.CompilerParams(use_tc_tiling_on_sc=False),
        scratch_types=[
            pltpu.VMEM((n_fields, nch, 128), jnp.int32),
            pltpu.VMEM((bpw, d), jnp.float32),
            pltpu.VMEM((bpw, _PAD), jnp.float32),
            pltpu.SemaphoreType.DMA,
            pltpu.SemaphoreType.DMA,
        ],
    )
    def body(x3_hbm, c_hbm, emb_hbm, s_hbm, zacc_hbm,
             idx_v, s_v, acc_v, sem, sem_c):
        cid = lax.axis_index("c")
        sid = lax.axis_index("s")
        wid = sid * _NC + cid
        base = wid * bpw
        cb = wid * nch

        idx_cps = [
            pltpu.async_copy(x3_hbm.at[f, pl.ds(cb, nch), :], idx_v.at[f], sem)
            for f in range(n_fields)
        ]

        zeros = jnp.zeros((_L,), jnp.float32)

        def zero_row(i, carry):
            for j in range(d // _L):
                s_v[i, pl.ds(j * _L, _L)] = zeros
            return carry

        lax.fori_loop(0, bpw, zero_row, 0)

        def zero_acc(i, carry):
            acc_v[i, pl.ds(0, _L)] = zeros
            return carry

        lax.fori_loop(0, bpw, zero_acc, 0)
        for cp in idx_cps:
            cp.wait()

        def drain_one():
            for ch in range(nch):
                pltpu.make_async_copy(
                    emb_hbm.at[pl.ds(0, 128)],
                    s_v.at[pl.ds(ch * 128, 128)], sem).wait()
                pltpu.make_async_copy(
                    c_hbm.at[pl.ds(0, 128)],
                    acc_v.at[pl.ds(ch * 128, 128)], sem_c).wait()

        def gather_field(f, carry):
            @pl.when(f > 0)
            def _():
                drain_one()
            for ch in range(nch):
                pltpu.async_copy(
                    emb_hbm.at[idx_v.at[f, ch]],
                    s_v.at[pl.ds(ch * 128, 128)],
                    sem, add=True)
                pltpu.async_copy(
                    c_hbm.at[idx_v.at[f, ch]],
                    acc_v.at[pl.ds(ch * 128, 128)],
                    sem_c, add=True)
            return carry

        lax.fori_loop(0, n_fields, gather_field, 0)
        drain_one()

        pltpu.sync_copy(s_v, s_hbm.at[pl.ds(base, bpw), :])
        pltpu.sync_copy(acc_v, zacc_hbm.at[pl.ds(base, bpw), :])

    return body(x3, c_tab, emb)


def _epilogue_body(s_ref, zacc_ref, out_ref):
    s = s_ref[...]
    z = zacc_ref[:, :1] + 0.5 * jnp.sum(s * s, axis=1, keepdims=True)
    out_ref[...] = 1.0 / (1.0 + jnp.exp(-z))


def _epilogue(s, zacc, *, batch, d):
    rb = 2048
    assert batch % rb == 0
    out = pl.pallas_call(
        _epilogue_body,
        grid=(batch // rb,),
        in_specs=[
            pl.BlockSpec((rb, d), lambda i: (i, 0)),
            pl.BlockSpec((rb, _PAD), lambda i: (i, 0)),
        ],
        out_specs=pl.BlockSpec((rb, 1), lambda i: (i, 0)),
        out_shape=jax.ShapeDtypeStruct((batch, 1), jnp.float32),
    )(s, zacc)
    return out.reshape(batch)


def kernel(x, fc_weight, bias, embedding_weight):
    batch, n_fields = x.shape
    v, d = embedding_weight.shape
    assert batch % (_NW * 128) == 0 and d % _L == 0

    c_pad = _make_combined_table(fc_weight, embedding_weight, bias, n_fields)
    v_pad = c_pad.shape[0]
    assert v_pad % (_NW * 8) == 0
    c_tab = _expand_c_kernel(c_pad, v=v_pad)
    x3 = x.astype(jnp.int32).T.reshape(n_fields, batch // 128, 128)
    s, zacc = _fm_sc_kernel(x3, c_tab, embedding_weight,
                            batch=batch, n_fields=n_fields, d=d)
    return _epilogue(s, zacc, batch=batch, d=d)


# speedup vs baseline: 1.5386x; 1.0048x over previous
import functools

import jax
import jax.numpy as jnp
from jax import lax
from jax.experimental import pallas as pl
from jax.experimental.pallas import tpu as pltpu
from jax.experimental.pallas import tpu_sc as plsc

_NC = 2
_NS = 16
_NW = _NC * _NS
_L = 16
_PAD = 16


def _combine_body(bias_ref, fc_ref, emb_ref, out_ref):
    e = emb_ref[...]
    out_ref[...] = (fc_ref[...] - 0.5 * jnp.sum(e * e, axis=1)
                    + bias_ref[0, 0])


def _make_combined_table(fc_weight, embedding_weight, bias, n_fields):
    v, d = embedding_weight.shape
    rb = 8192
    n_blocks = -(-v // rb)
    v_pad = n_blocks * rb
    bias_over_f = (bias.astype(jnp.float32) / n_fields).reshape(1, 1)
    return pl.pallas_call(
        _combine_body,
        grid=(n_blocks,),
        in_specs=[
            pl.BlockSpec(memory_space=pltpu.SMEM),
            pl.BlockSpec((rb,), lambda i: (i,)),
            pl.BlockSpec((rb, d), lambda i: (i, 0)),
        ],
        out_specs=pl.BlockSpec((rb,), lambda i: (i,)),
        out_shape=jax.ShapeDtypeStruct((v_pad,), jnp.float32),
    )(bias_over_f, fc_weight.reshape(v), embedding_weight)


def _expand_c_kernel(c, *, v):
    rows_pw = v // _NW
    assert rows_pw % 8 == 0
    mesh = plsc.VectorSubcoreMesh(
        core_axis_name="c", subcore_axis_name="s",
        num_cores=_NC, num_subcores=_NS,
    )

    @functools.partial(
        pl.kernel,
        out_type=jax.ShapeDtypeStruct((v, _PAD), jnp.float32),
        mesh=mesh,
        compiler_params=pltpu.CompilerParams(use_tc_tiling_on_sc=False),
        scratch_types=[
            pltpu.VMEM((rows_pw,), jnp.float32),
            pltpu.VMEM((rows_pw, _PAD), jnp.float32),
            pltpu.SemaphoreType.DMA,
        ],
    )
    def body(c_hbm, out_hbm, c_v, e_v, sem):
        cid = lax.axis_index("c")
        sid = lax.axis_index("s")
        wid = sid * _NC + cid
        base = wid * rows_pw

        pltpu.async_copy(c_hbm.at[pl.ds(base, rows_pw)], c_v, sem).wait()

        assert rows_pw % _L == 0

        def expand_group(g, carry):
            vals = c_v[pl.ds(g * _L, _L)]
            for k in range(_L):
                e_v[g * _L + k, pl.ds(0, _L)] = jnp.full(
                    (_L,), vals[k], jnp.float32)
            return carry

        lax.fori_loop(0, rows_pw // _L, expand_group, 0)
        pltpu.sync_copy(e_v, out_hbm.at[pl.ds(base, rows_pw), :])

    return body(c)


def _fm_sc_kernel(x3, c_tab, emb, *, batch, n_fields, d):
    bpw = batch // _NW
    nch = bpw // 128
    mesh = plsc.VectorSubcoreMesh(
        core_axis_name="c", subcore_axis_name="s",
        num_cores=_NC, num_subcores=_NS,
    )

    @functools.partial(
        pl.kernel,
        out_type=(
            jax.ShapeDtypeStruct((batch, d), jnp.float32),
            jax.ShapeDtypeStruct((batch, _PAD), jnp.float32),
        ),
        mesh=mesh,
        compiler_params=pltpu.CompilerParams(use_tc_tiling_on_sc=False),
        scratch_types=[
            pltpu.VMEM((n_fields, nch, 128), jnp.int32),
            pltpu.VMEM((bpw, d), jnp.float32),
            pltpu.VMEM((bpw, _PAD), jnp.float32),
            pltpu.SemaphoreType.DMA,
            pltpu.SemaphoreType.DMA,
        ],
    )
    def body(x3_hbm, c_hbm, emb_hbm, s_hbm, zacc_hbm,
             idx_v, s_v, acc_v, sem, sem_c):
        cid = lax.axis_index("c")
        sid = lax.axis_index("s")
        wid = sid * _NC + cid
        base = wid * bpw
        cb = wid * nch

        idx_cps = [
            pltpu.async_copy(x3_hbm.at[f, pl.ds(cb, nch), :], idx_v.at[f], sem)
            for f in range(n_fields)
        ]

        zeros = jnp.zeros((_L,), jnp.float32)

        def zero_row(i, carry):
            for j in range(d // _L):
                s_v[i, pl.ds(j * _L, _L)] = zeros
            return carry

        lax.fori_loop(0, bpw, zero_row, 0)

        def zero_acc(i, carry):
            acc_v[i, pl.ds(0, _L)] = zeros
            return carry

        lax.fori_loop(0, bpw, zero_acc, 0)
        for cp in idx_cps:
            cp.wait()

        def drain_one():
            for ch in range(nch):
                pltpu.make_async_copy(
                    emb_hbm.at[pl.ds(0, 128)],
                    s_v.at[pl.ds(ch * 128, 128)], sem).wait()
                pltpu.make_async_copy(
                    c_hbm.at[pl.ds(0, 128)],
                    acc_v.at[pl.ds(ch * 128, 128)], sem_c).wait()

        def gather_field(f, carry):
            @pl.when(f > 0)
            def _():
                drain_one()
            for ch in range(nch):
                pltpu.async_copy(
                    emb_hbm.at[idx_v.at[f, ch]],
                    s_v.at[pl.ds(ch * 128, 128)],
                    sem, add=True)
                pltpu.async_copy(
                    c_hbm.at[idx_v.at[f, ch]],
                    acc_v.at[pl.ds(ch * 128, 128)],
                    sem_c, add=True)
            return carry

        lax.fori_loop(0, n_fields, gather_field, 0)
        drain_one()

        pltpu.sync_copy(s_v, s_hbm.at[pl.ds(base, bpw), :])
        pltpu.sync_copy(acc_v, zacc_hbm.at[pl.ds(base, bpw), :])

    return body(x3, c_tab, emb)


def _epilogue_body(s_ref, zacc_ref, out_ref):
    s = s_ref[...]
    z = zacc_ref[:, :1] + 0.5 * jnp.sum(s * s, axis=1, keepdims=True)
    out_ref[...] = 1.0 / (1.0 + jnp.exp(-z))


def _epilogue(s, zacc, *, batch, d):
    rb = 2048
    assert batch % rb == 0
    out = pl.pallas_call(
        _epilogue_body,
        grid=(batch // rb,),
        in_specs=[
            pl.BlockSpec((rb, d), lambda i: (i, 0)),
            pl.BlockSpec((rb, _PAD), lambda i: (i, 0)),
        ],
        out_specs=pl.BlockSpec((rb, 1), lambda i: (i, 0)),
        out_shape=jax.ShapeDtypeStruct((batch, 1), jnp.float32),
    )(s, zacc)
    return out.reshape(batch)


def kernel(x, fc_weight, bias, embedding_weight):
    batch, n_fields = x.shape
    v, d = embedding_weight.shape
    assert batch % (_NW * 128) == 0 and d % _L == 0

    c_pad = _make_combined_table(fc_weight, embedding_weight, bias, n_fields)
    v_pad = c_pad.shape[0]
    assert v_pad % (_NW * 8) == 0
    c_tab = _expand_c_kernel(c_pad, v=v_pad)
    x3 = x.astype(jnp.int32).T.reshape(n_fields, batch // 128, 128)
    s, zacc = _fm_sc_kernel(x3, c_tab, embedding_weight,
                            batch=batch, n_fields=n_fields, d=d)
    return _epilogue(s, zacc, batch=batch, d=d)
